# Initial kernel scaffold; baseline (speedup 1.0000x reference)
import jax
import jax.numpy as jnp
from jax.experimental import pallas as pl


def kernel(x, pe):
    raise NotImplementedError("write your pallas kernel here")


# speedup vs baseline: 2.7720x; 2.7720x over previous
import jax
import jax.numpy as jnp
from jax.experimental import pallas as pl


def _copy_body(pe_ref, out_ref):
    out_ref[0, :, :] = pe_ref[...]


def kernel(x, pe):
    seq_len = x.shape[1]
    n_emb = pe.shape[1]
    block_rows = 512
    grid = (seq_len // block_rows,)
    out = pl.pallas_call(
        _copy_body,
        grid=grid,
        in_specs=[pl.BlockSpec((block_rows, n_emb), lambda i: (i, 0))],
        out_specs=pl.BlockSpec((1, block_rows, n_emb), lambda i: (0, i, 0)),
        out_shape=jax.ShapeDtypeStruct((1, seq_len, n_emb), pe.dtype),
    )(pe)
    return out
